# baseline (speedup 1.0000x reference)
import functools

import jax
import jax.numpy as jnp
from jax import lax
from jax.experimental import pallas as pl
from jax.experimental.pallas import tpu as pltpu
from jax.experimental.pallas import tpu_sc as plsc

L = 16
KB = 64
WB = 2000
VR = 128


@functools.lru_cache(maxsize=None)
def _make_update(M, D, B, num_cores=2, num_subcores=16):
  NW = num_cores * num_subcores
  NB = M // WB
  assert M % WB == 0 and B % L == 0 and L < D <= 2 * L
  DHI = D - L
  NVEC = B // L

  mesh = plsc.VectorSubcoreMesh(
      core_axis_name="c", subcore_axis_name="s",
      num_cores=num_cores, num_subcores=num_subcores)

  lanes = lambda: lax.iota(jnp.int32, L)
  sp = lambda x: jnp.full((L,), x, jnp.int32)

  def body(memf_hbm, valr_hbm, idx_hbm, outf_hbm,
           idx_v, jl, bl, rbl, lmbuf, offbuf, vbuf, buf, sem, semb):
    wid = lax.axis_index("s") * num_cores + lax.axis_index("c")

    pltpu.sync_copy(idx_hbm, idx_v)

    SU = 32

    def scan_body(t, cnt):
      for i8 in range(SU):
        v = idx_v[pl.ds(t * (SU * L) + i8 * L, L)]
        own = lax.rem(lax.div(v, sp(WB)), sp(NW)) == sp(wid)
        pos = lanes() + sp(t * (SU * L) + i8 * L)
        offs = plsc.cumsum(own.astype(jnp.int32)) - 1
        plsc.store_scatter(jl, [sp(cnt) + offs], pos, mask=own)
        cnt = cnt + jnp.sum(own.astype(jnp.int32))
      return cnt
    cnt = lax.fori_loop(0, NVEC // SU, scan_body, jnp.int32(0))

    def block_body(bi, _):
      b = wid + bi * NW
      m0 = b * jnp.int32(WB)
      hin = [pltpu.async_copy(memf_hbm.at[pl.ds(d * M + m0, WB)],
                              buf.at[pl.ds(d * WB, WB)], semb)
             for d in range(D)]
      for h in hin:
        h.wait()

      def bscan(t, bcnt):
        for i8 in range(SU):
          pos = lanes() + sp(t * (SU * L) + i8 * L)
          valid = pos < sp(cnt)
          j = jl[pl.ds(t * (SU * L) + i8 * L, L)]
          jc = jnp.where(valid, j, sp(0))
          m = plsc.load_gather(idx_v, [jc])
          inb = valid & (m >= sp(m0)) & (m < sp(m0) + sp(WB))
          offs = plsc.cumsum(inb.astype(jnp.int32)) - 1
          pk = jc * sp(2048) + (m - sp(m0))
          plsc.store_scatter(bl, [sp(bcnt) + offs], pk, mask=inb)
          bcnt = bcnt + jnp.sum(inb.astype(jnp.int32))
        return bcnt
      nvt = lax.div(cnt + jnp.int32(SU * L - 1), jnp.int32(SU * L))
      bcnt = lax.fori_loop(0, nvt, bscan, jnp.int32(0))

      def chunk_body(q, _):
        base = q * KB
        for i in range(KB // L):
          pos = lanes() + sp(base + i * L)
          valid = pos < sp(bcnt)
          pk = bl[pl.ds(base + i * L, L)]
          j = lax.div(pk, sp(2048))
          lm = pk - j * sp(2048)
          lmbuf[pl.ds(i * L, L)] = lm
          s = jnp.where(valid, j, sp(0)) * sp(D)
          r0 = lax.div(s, sp(VR))
          off = s - r0 * sp(VR)
          offbuf[pl.ds(i * L, L)] = off
          r1 = r0 + (off > sp(VR - D)).astype(jnp.int32)
          ppos = (lanes() + sp(i * L)) * sp(2)
          plsc.store_scatter(rbl, [ppos], r0)
          plsc.store_scatter(rbl, [ppos + sp(1)], r1)
        pltpu.async_copy(valr_hbm.at[rbl], vbuf, sem).wait()

        for i in range(KB // L):
          pos = lanes() + sp(base + i * L)
          valid = pos < sp(bcnt)
          lmv = lmbuf[pl.ds(i * L, L)]
          offv = offbuf[pl.ds(i * L, L)]
          rowb = (lanes() + sp(i * L)) * sp(2)
          for c in range(D):
            col = offv + sp(c)
            rs = (col >= sp(VR)).astype(jnp.int32)
            vd = plsc.load_gather(
                vbuf, [rowb + rs, col - rs * sp(VR)], mask=valid)
            plsc.addupdate_scatter(
                buf, [sp(c * WB) + lmv], vd, mask=valid)
        return 0
      nq = lax.div(bcnt + jnp.int32(KB - 1), jnp.int32(KB))
      lax.fori_loop(0, nq, chunk_body, 0)

      hout = [pltpu.async_copy(buf.at[pl.ds(d * WB, WB)],
                               outf_hbm.at[pl.ds(d * M + m0, WB)], semb)
              for d in range(D)]
      for h in hout:
        h.wait()
      return 0

    nb_w = lax.div(jnp.int32(NB) - wid + jnp.int32(NW - 1), jnp.int32(NW))
    lax.fori_loop(0, nb_w, block_body, 0)

  return pl.kernel(
      body,
      out_type=jax.ShapeDtypeStruct((D * M,), jnp.float32),
      mesh=mesh,
      scratch_types=[
          pltpu.VMEM((B,), jnp.int32),
          pltpu.VMEM((B + L,), jnp.int32),
          pltpu.VMEM((B + L,), jnp.int32),
          pltpu.VMEM((2 * KB,), jnp.int32),
          pltpu.VMEM((KB,), jnp.int32),
          pltpu.VMEM((KB,), jnp.int32),
          pltpu.VMEM((2 * KB, VR), jnp.float32),
          pltpu.VMEM((D * WB,), jnp.float32),
          pltpu.SemaphoreType.DMA,
          pltpu.SemaphoreType.DMA,
      ],
      compiler_params=pltpu.CompilerParams(needs_layout_passes=False),
  )


def kernel(mem, val, idx):
  M, D = mem.shape
  B = val.shape[0]
  idx32 = idx.astype(jnp.int32)
  memf = mem.T.reshape(D * M)
  valr = val.reshape(B * D // VR, VR)
  outf = _make_update(M, D, B)(memf, valr, idx32)
  return outf.reshape(D, M).T


# speedup vs baseline: 2.2151x; 2.2151x over previous
import functools

import jax
import jax.numpy as jnp
from jax import lax
from jax.experimental import pallas as pl
from jax.experimental.pallas import tpu as pltpu
from jax.experimental.pallas import tpu_sc as plsc

L = 16
KB = 64
WB = 2000
VR = 128


@functools.lru_cache(maxsize=None)
def _make_update(M, D, B, num_cores=2, num_subcores=16):
  NW = num_cores * num_subcores
  NB = M // WB
  assert M % WB == 0 and B % L == 0 and L < D <= 2 * L
  DHI = D - L
  NVEC = B // L

  mesh = plsc.VectorSubcoreMesh(
      core_axis_name="c", subcore_axis_name="s",
      num_cores=num_cores, num_subcores=num_subcores)

  lanes = lambda: lax.iota(jnp.int32, L)
  sp = lambda x: jnp.full((L,), x, jnp.int32)

  def body(memf_hbm, valr_hbm, idx_hbm, outf_hbm,
           idx_v, jl, bl, rbl, lmbuf, offbuf, vbuf, buf, sem, semb):
    wid = lax.axis_index("s") * num_cores + lax.axis_index("c")

    pltpu.sync_copy(idx_hbm, idx_v)

    SU = 32

    def scan_body(t, cnt):
      for i8 in range(SU):
        v = idx_v[pl.ds(t * (SU * L) + i8 * L, L)]
        own = lax.rem(lax.div(v, sp(WB)), sp(NW)) == sp(wid)
        pos = lanes() + sp(t * (SU * L) + i8 * L)
        offs = plsc.cumsum(own.astype(jnp.int32)) - 1
        plsc.store_scatter(jl, [sp(cnt) + offs], pos, mask=own)
        cnt = cnt + jnp.sum(own.astype(jnp.int32))
      return cnt
    cnt = lax.fori_loop(0, NVEC // SU, scan_body, jnp.int32(0))

    def block_body(bi, _):
      b = wid + bi * NW
      m0 = b * jnp.int32(WB)
      pltpu.async_copy(memf_hbm.at[pl.ds(m0 * D, WB * D)], buf, semb).wait()

      def bscan(t, bcnt):
        for i8 in range(SU):
          pos = lanes() + sp(t * (SU * L) + i8 * L)
          valid = pos < sp(cnt)
          j = jl[pl.ds(t * (SU * L) + i8 * L, L)]
          jc = jnp.where(valid, j, sp(0))
          m = plsc.load_gather(idx_v, [jc])
          inb = valid & (m >= sp(m0)) & (m < sp(m0) + sp(WB))
          offs = plsc.cumsum(inb.astype(jnp.int32)) - 1
          pk = jc * sp(2048) + (m - sp(m0))
          plsc.store_scatter(bl, [sp(bcnt) + offs], pk, mask=inb)
          bcnt = bcnt + jnp.sum(inb.astype(jnp.int32))
        return bcnt
      nvt = lax.div(cnt + jnp.int32(SU * L - 1), jnp.int32(SU * L))
      bcnt = lax.fori_loop(0, nvt, bscan, jnp.int32(0))

      def chunk_body(q, _):
        base = q * KB
        for i in range(KB // L):
          pos = lanes() + sp(base + i * L)
          valid = pos < sp(bcnt)
          pk = bl[pl.ds(base + i * L, L)]
          j = lax.div(pk, sp(2048))
          lm = pk - j * sp(2048)
          lmbuf[pl.ds(i * L, L)] = lm
          s = jnp.where(valid, j, sp(0)) * sp(D)
          r0 = lax.div(s, sp(VR))
          off = s - r0 * sp(VR)
          offbuf[pl.ds(i * L, L)] = off
          r1 = r0 + (off > sp(VR - D)).astype(jnp.int32)
          ppos = (lanes() + sp(i * L)) * sp(2)
          plsc.store_scatter(rbl, [ppos], r0)
          plsc.store_scatter(rbl, [ppos + sp(1)], r1)
        pltpu.async_copy(valr_hbm.at[rbl], vbuf, sem).wait()

        for i in range(KB // L):
          pos = lanes() + sp(base + i * L)
          valid = pos < sp(bcnt)
          lmv = lmbuf[pl.ds(i * L, L)]
          offv = offbuf[pl.ds(i * L, L)]
          rowb = (lanes() + sp(i * L)) * sp(2)
          for c in range(D):
            col = offv + sp(c)
            rs = (col >= sp(VR)).astype(jnp.int32)
            vd = plsc.load_gather(
                vbuf, [rowb + rs, col - rs * sp(VR)], mask=valid)
            plsc.addupdate_scatter(
                buf, [lmv * sp(D) + sp(c)], vd, mask=valid)
        return 0
      nq = lax.div(bcnt + jnp.int32(KB - 1), jnp.int32(KB))
      lax.fori_loop(0, nq, chunk_body, 0)

      pltpu.async_copy(buf, outf_hbm.at[pl.ds(m0 * D, WB * D)], semb).wait()
      return 0

    nb_w = lax.div(jnp.int32(NB) - wid + jnp.int32(NW - 1), jnp.int32(NW))
    lax.fori_loop(0, nb_w, block_body, 0)

  return pl.kernel(
      body,
      out_type=jax.ShapeDtypeStruct((D * M,), jnp.float32),
      mesh=mesh,
      scratch_types=[
          pltpu.VMEM((B,), jnp.int32),
          pltpu.VMEM((B + L,), jnp.int32),
          pltpu.VMEM((B + L,), jnp.int32),
          pltpu.VMEM((2 * KB,), jnp.int32),
          pltpu.VMEM((KB,), jnp.int32),
          pltpu.VMEM((KB,), jnp.int32),
          pltpu.VMEM((2 * KB, VR), jnp.float32),
          pltpu.VMEM((D * WB,), jnp.float32),
          pltpu.SemaphoreType.DMA,
          pltpu.SemaphoreType.DMA,
      ],
      compiler_params=pltpu.CompilerParams(needs_layout_passes=False),
  )


def kernel(mem, val, idx):
  M, D = mem.shape
  B = val.shape[0]
  idx32 = idx.astype(jnp.int32)
  memf = mem.reshape(M * D)
  valr = val.reshape(B * D // VR, VR)
  outf = _make_update(M, D, B)(memf, valr, idx32)
  return outf.reshape(M, D)
